# baseline (speedup 1.0000x reference)
import jax
from jax.experimental import pallas as pl
from jax.experimental.pallas import tpu as pltpu


def _copy_body(f_in, ei_in, ea_in, f_out, ei_out, ea_out, sem):
    c0 = pltpu.make_async_copy(f_in, f_out, sem.at[0])
    c1 = pltpu.make_async_copy(ei_in, ei_out, sem.at[1])
    c2 = pltpu.make_async_copy(ea_in, ea_out, sem.at[2])
    c0.start()
    c1.start()
    c2.start()
    c0.wait()
    c1.wait()
    c2.wait()


def kernel(feats, edge_index, edge_attr):
    return pl.pallas_call(
        _copy_body,
        in_specs=[pl.BlockSpec(memory_space=pl.ANY)] * 3,
        out_specs=[pl.BlockSpec(memory_space=pl.ANY)] * 3,
        out_shape=[
            jax.ShapeDtypeStruct(feats.shape, feats.dtype),
            jax.ShapeDtypeStruct(edge_index.shape, edge_index.dtype),
            jax.ShapeDtypeStruct(edge_attr.shape, edge_attr.dtype),
        ],
        scratch_shapes=[pltpu.SemaphoreType.DMA((3,))],
    )(feats, edge_index, edge_attr)


# speedup vs baseline: 13.8879x; 13.8879x over previous
import jax
import jax.numpy as jnp
from jax.experimental import pallas as pl
from jax.experimental.pallas import tpu as pltpu

_GRID = 5
_LANES = 128


def _copy_body(f_in, ei_in, ea_in, f_out, ei_out, ea_out):
    f_out[...] = f_in[...]
    ei_out[...] = ei_in[...]
    ea_out[...] = ea_in[...]


def kernel(feats, edge_index, edge_attr):
    n, d = feats.shape
    e, ik = edge_index.shape
    _, ak = edge_attr.shape

    ei2 = edge_index.reshape(e * ik // _LANES, _LANES)
    ea2 = edge_attr.reshape(e * ak // _LANES, _LANES)

    bf = n // _GRID
    bi = ei2.shape[0] // _GRID
    ba = ea2.shape[0] // _GRID

    f_o, ei_o, ea_o = pl.pallas_call(
        _copy_body,
        grid=(_GRID,),
        in_specs=[
            pl.BlockSpec((bf, d), lambda i: (i, 0)),
            pl.BlockSpec((bi, _LANES), lambda i: (i, 0)),
            pl.BlockSpec((ba, _LANES), lambda i: (i, 0)),
        ],
        out_specs=[
            pl.BlockSpec((bf, d), lambda i: (i, 0)),
            pl.BlockSpec((bi, _LANES), lambda i: (i, 0)),
            pl.BlockSpec((ba, _LANES), lambda i: (i, 0)),
        ],
        out_shape=[
            jax.ShapeDtypeStruct(feats.shape, feats.dtype),
            jax.ShapeDtypeStruct(ei2.shape, edge_index.dtype),
            jax.ShapeDtypeStruct(ea2.shape, edge_attr.dtype),
        ],
        compiler_params=pltpu.CompilerParams(
            dimension_semantics=("arbitrary",),
        ),
    )(feats, ei2, ea2)

    return (f_o, ei_o.reshape(e, ik), ea_o.reshape(e, ak))


# speedup vs baseline: 19.5356x; 1.4067x over previous
import jax
import jax.numpy as jnp
from jax.experimental import pallas as pl
from jax.experimental.pallas import tpu as pltpu

_GRID = 50


def _copy_body(f_in, ei_in, ea_in, f_out, ei_out, ea_out):
    f_out[...] = f_in[...]
    ei_out[...] = ei_in[...]
    ea_out[...] = ea_in[...]


def kernel(feats, edge_index, edge_attr):
    n, d = feats.shape
    e, ik = edge_index.shape
    _, ak = edge_attr.shape

    bf = n // _GRID
    be = e // _GRID

    f_o, ei_o, ea_o = pl.pallas_call(
        _copy_body,
        grid=(_GRID,),
        in_specs=[
            pl.BlockSpec((bf, d), lambda i: (i, 0)),
            pl.BlockSpec((be, ik), lambda i: (i, 0)),
            pl.BlockSpec((be, ak), lambda i: (i, 0)),
        ],
        out_specs=[
            pl.BlockSpec((bf, d), lambda i: (i, 0)),
            pl.BlockSpec((be, ik), lambda i: (i, 0)),
            pl.BlockSpec((be, ak), lambda i: (i, 0)),
        ],
        out_shape=[
            jax.ShapeDtypeStruct(feats.shape, feats.dtype),
            jax.ShapeDtypeStruct(edge_index.shape, edge_index.dtype),
            jax.ShapeDtypeStruct(edge_attr.shape, edge_attr.dtype),
        ],
        compiler_params=pltpu.CompilerParams(
            dimension_semantics=("arbitrary",),
        ),
    )(feats, edge_index, edge_attr)

    return (f_o, ei_o, ea_o)
